# baseline (speedup 1.0000x reference)
import functools

import jax
import jax.numpy as jnp
from jax import lax
from jax.experimental import pallas as pl
from jax.experimental.pallas import tpu as pltpu
from jax.experimental.pallas import tpu_sc as plsc

N_NODES = 10000
D = 128
N_EDGES = 320000

NC = 2
NS = 16
NW = NC * NS

CHUNK = 128
ACC_ROWS = 10240
ROWS_PER_TILE = ACC_ROWS // NS
CHUNKS_PER_W = 80
EDGES_PAD = NW * CHUNK * CHUNKS_PER_W
N_DUMP = ACC_ROWS - N_NODES


def _sc_segment_sum(x, src_w, dst_w, zrows, zdeg):
    mesh = plsc.VectorSubcoreMesh(
        core_axis_name="c", subcore_axis_name="s", num_cores=NC, num_subcores=NS
    )

    @functools.partial(
        pl.kernel,
        mesh=mesh,
        out_type=(
            jax.ShapeDtypeStruct((NC, ACC_ROWS, D), jnp.float32),
            jax.ShapeDtypeStruct((NC, ACC_ROWS), jnp.float32),
        ),
        scratch_types=[
            pltpu.VMEM_SHARED((ACC_ROWS, D), jnp.float32),
            pltpu.VMEM_SHARED((ACC_ROWS,), jnp.float32),
            pltpu.VMEM((CHUNKS_PER_W, CHUNK), jnp.int32),
            pltpu.VMEM((CHUNKS_PER_W, CHUNK), jnp.int32),
            pltpu.VMEM((CHUNK, D), jnp.float32),
            pltpu.VMEM((CHUNK,), jnp.float32),
        ],
    )
    def seg_sum(x_hbm, src_hbm, dst_hbm, zr_hbm, zd_hbm, part_hbm, deg_hbm,
                acc, dacc, srcv, dstv, rows0, ones):
        cid = lax.axis_index("c")
        sid = lax.axis_index("s")
        wid = sid * NC + cid
        base = sid * ROWS_PER_TILE

        pltpu.sync_copy(zr_hbm.at[pl.ds(base, ROWS_PER_TILE)],
                        acc.at[pl.ds(base, ROWS_PER_TILE)])
        pltpu.sync_copy(zd_hbm.at[pl.ds(base, ROWS_PER_TILE)],
                        dacc.at[pl.ds(base, ROWS_PER_TILE)])
        for i in range(CHUNK // 16):
            ones[pl.ds(i * 16, 16)] = jnp.full((16,), 1.0, jnp.float32)

        pltpu.sync_copy(src_hbm.at[wid], srcv)
        pltpu.sync_copy(dst_hbm.at[wid], dstv)
        plsc.subcore_barrier()

        def body(j, _):
            pltpu.sync_copy(x_hbm.at[srcv.at[j]], rows0)
            pltpu.sync_copy(rows0, acc.at[dstv.at[j]], add=True)
            pltpu.sync_copy(ones, dacc.at[dstv.at[j]], add=True)
            return ()

        lax.fori_loop(0, CHUNKS_PER_W, body, ())
        plsc.subcore_barrier()

        pltpu.sync_copy(acc.at[pl.ds(base, ROWS_PER_TILE)],
                        part_hbm.at[cid].at[pl.ds(base, ROWS_PER_TILE)])
        pltpu.sync_copy(dacc.at[pl.ds(base, ROWS_PER_TILE)],
                        deg_hbm.at[cid].at[pl.ds(base, ROWS_PER_TILE)])

    return seg_sum(x, src_w, dst_w, zrows, zdeg)


ROW_BLK = 1000


def _tc_body(part_ref0, part_ref1, x_ref, wl_ref, wr_ref, bl_ref, br_ref,
             d0_ref, d1_ref, o_ref):
    agg = part_ref0[0] + part_ref1[0]
    acc = jnp.dot(agg, wl_ref[...], preferred_element_type=jnp.float32)
    acc += jnp.dot(x_ref[...], wr_ref[...], preferred_element_type=jnp.float32)
    acc += (d0_ref[0] + d1_ref[0]) * bl_ref[...]
    acc += br_ref[...]
    o_ref[...] = jnp.maximum(acc, 0.0)


def _tc_epilogue(part, x, wl_t, wr_t, bl, br, degp):
    grid = (N_NODES // ROW_BLK,)
    blk = lambda i: (i, 0)
    full = lambda i: (0, 0)
    return pl.pallas_call(
        _tc_body,
        grid=grid,
        in_specs=[
            pl.BlockSpec((1, ROW_BLK, D), lambda i: (0, i, 0)),
            pl.BlockSpec((1, ROW_BLK, D), lambda i: (1, i, 0)),
            pl.BlockSpec((ROW_BLK, D), blk),
            pl.BlockSpec((D, D), full),
            pl.BlockSpec((D, D), full),
            pl.BlockSpec((1, D), full),
            pl.BlockSpec((1, D), full),
            pl.BlockSpec((1, ROW_BLK, 1), lambda i: (0, i, 0)),
            pl.BlockSpec((1, ROW_BLK, 1), lambda i: (1, i, 0)),
        ],
        out_specs=pl.BlockSpec((ROW_BLK, D), blk),
        out_shape=jax.ShapeDtypeStruct((N_NODES, D), jnp.float32),
    )(part, part, x, wl_t, wr_t, bl, br, degp, degp)


@jax.jit
def kernel(x, edge_index, W_lin, b_lin, W_root, b_root):
    src = edge_index[0].astype(jnp.int32)
    dst = edge_index[1].astype(jnp.int32)
    pad = EDGES_PAD - N_EDGES
    src_w = jnp.concatenate([src, jnp.zeros((pad,), jnp.int32)])
    dst_w = jnp.concatenate(
        [dst, N_NODES + (jnp.arange(pad, dtype=jnp.int32) % N_DUMP)])
    src_w = src_w.reshape(NW, CHUNKS_PER_W, CHUNK)
    dst_w = dst_w.reshape(NW, CHUNKS_PER_W, CHUNK)
    zrows = jnp.zeros((ACC_ROWS, D), jnp.float32)
    zdeg = jnp.zeros((ACC_ROWS,), jnp.float32)

    part, degp = _sc_segment_sum(x, src_w, dst_w, zrows, zdeg)

    out = _tc_epilogue(
        part, x, W_lin.T, W_root.T,
        b_lin.reshape(1, D), b_root.reshape(1, D),
        degp.reshape(NC, ACC_ROWS, 1),
    )
    return out


# speedup vs baseline: 2.5379x; 2.5379x over previous
import functools

import jax
import jax.numpy as jnp
from jax import lax
from jax.experimental import pallas as pl
from jax.experimental.pallas import tpu as pltpu
from jax.experimental.pallas import tpu_sc as plsc

N_NODES = 10000
D = 128
N_EDGES = 320000

NC = 2
NS = 16
NW = NC * NS

CHUNK = 128
ACC_ROWS = 10240
ROWS_PER_TILE = ACC_ROWS // NS
CHUNKS_PER_W = 80
EDGES_PAD = NW * CHUNK * CHUNKS_PER_W
N_DUMP = ACC_ROWS - N_NODES


def _sc_segment_sum(x, src_w, dst_w, zrows, zdeg):
    mesh = plsc.VectorSubcoreMesh(
        core_axis_name="c", subcore_axis_name="s", num_cores=NC, num_subcores=NS
    )

    @functools.partial(
        pl.kernel,
        mesh=mesh,
        out_type=(
            jax.ShapeDtypeStruct((NC, ACC_ROWS, D), jnp.float32),
            jax.ShapeDtypeStruct((NC, ACC_ROWS), jnp.float32),
        ),
        scratch_types=[
            pltpu.VMEM_SHARED((ACC_ROWS, D), jnp.float32),
            pltpu.VMEM_SHARED((ACC_ROWS,), jnp.float32),
            pltpu.VMEM((CHUNKS_PER_W, CHUNK), jnp.int32),
            pltpu.VMEM((CHUNKS_PER_W, CHUNK), jnp.int32),
            pltpu.VMEM((CHUNK, D), jnp.float32),
            pltpu.VMEM((CHUNK,), jnp.float32),
        ],
    )
    def seg_sum(x_hbm, src_hbm, dst_hbm, zr_hbm, zd_hbm, part_hbm, deg_hbm,
                acc, dacc, srcv, dstv, rows0, ones):
        cid = lax.axis_index("c")
        sid = lax.axis_index("s")
        wid = sid * NC + cid
        base = sid * ROWS_PER_TILE

        pltpu.sync_copy(zr_hbm.at[pl.ds(base, ROWS_PER_TILE)],
                        acc.at[pl.ds(base, ROWS_PER_TILE)])
        pltpu.sync_copy(zd_hbm.at[pl.ds(base, ROWS_PER_TILE)],
                        dacc.at[pl.ds(base, ROWS_PER_TILE)])
        for i in range(CHUNK // 16):
            ones[pl.ds(i * 16, 16)] = jnp.full((16,), 1.0, jnp.float32)

        pltpu.sync_copy(src_hbm.at[wid], srcv)
        pltpu.sync_copy(dst_hbm.at[wid], dstv)
        plsc.subcore_barrier()

        def body(j, _):
            pltpu.sync_copy(x_hbm.at[srcv.at[j]], rows0)
            pltpu.sync_copy(rows0, acc.at[dstv.at[j]], add=True)
            pltpu.sync_copy(ones, dacc.at[dstv.at[j]], add=True)
            return ()

        lax.fori_loop(0, CHUNKS_PER_W, body, ())
        plsc.subcore_barrier()

        pltpu.sync_copy(acc.at[pl.ds(base, ROWS_PER_TILE)],
                        part_hbm.at[cid].at[pl.ds(base, ROWS_PER_TILE)])
        pltpu.sync_copy(dacc.at[pl.ds(base, ROWS_PER_TILE)],
                        deg_hbm.at[cid].at[pl.ds(base, ROWS_PER_TILE)])

    return seg_sum(x, src_w, dst_w, zrows, zdeg)


ROW_BLK = 1000


def _tc_body(part_ref0, part_ref1, x_ref, wl_ref, wr_ref, bl_ref, br_ref,
             d0_ref, d1_ref, o_ref):
    agg = part_ref0[0] + part_ref1[0]
    acc = jnp.dot(agg, wl_ref[...], preferred_element_type=jnp.float32)
    acc += jnp.dot(x_ref[...], wr_ref[...], preferred_element_type=jnp.float32)
    acc += (d0_ref[0] + d1_ref[0]) * bl_ref[...]
    acc += br_ref[...]
    o_ref[...] = jnp.maximum(acc, 0.0)


def _tc_epilogue(part, x, wl_t, wr_t, bl, br, degp):
    grid = (N_NODES // ROW_BLK,)
    blk = lambda i: (i, 0)
    full = lambda i: (0, 0)
    return pl.pallas_call(
        _tc_body,
        grid=grid,
        in_specs=[
            pl.BlockSpec((1, ROW_BLK, D), lambda i: (0, i, 0)),
            pl.BlockSpec((1, ROW_BLK, D), lambda i: (1, i, 0)),
            pl.BlockSpec((ROW_BLK, D), blk),
            pl.BlockSpec((D, D), full),
            pl.BlockSpec((D, D), full),
            pl.BlockSpec((1, D), full),
            pl.BlockSpec((1, D), full),
            pl.BlockSpec((1, ROW_BLK, 1), lambda i: (0, i, 0)),
            pl.BlockSpec((1, ROW_BLK, 1), lambda i: (1, i, 0)),
        ],
        out_specs=pl.BlockSpec((ROW_BLK, D), blk),
        out_shape=jax.ShapeDtypeStruct((N_NODES, D), jnp.float32),
    )(part, part, x, wl_t, wr_t, bl, br, degp, degp)


@jax.jit
def kernel(x, edge_index, W_lin, b_lin, W_root, b_root):
    src = edge_index[0].astype(jnp.int32)
    dst = edge_index[1].astype(jnp.int32)
    pad = EDGES_PAD - N_EDGES
    src_w = jnp.concatenate(
        [src, jnp.arange(pad, dtype=jnp.int32) % N_NODES])
    dst_w = jnp.concatenate(
        [dst, N_NODES + (jnp.arange(pad, dtype=jnp.int32) % N_DUMP)])
    src_w = src_w.reshape(NW, CHUNKS_PER_W, CHUNK)
    dst_w = dst_w.reshape(NW, CHUNKS_PER_W, CHUNK)
    zrows = jnp.zeros((ACC_ROWS, D), jnp.float32)
    zdeg = jnp.zeros((ACC_ROWS,), jnp.float32)

    part, degp = _sc_segment_sum(x, src_w, dst_w, zrows, zdeg)

    out = _tc_epilogue(
        part, x, W_lin.T, W_root.T,
        b_lin.reshape(1, D), b_root.reshape(1, D),
        degp.reshape(NC, ACC_ROWS, 1),
    )
    return out


# speedup vs baseline: 3.6164x; 1.4250x over previous
import functools

import jax
import jax.numpy as jnp
from jax import lax
from jax.experimental import pallas as pl
from jax.experimental.pallas import tpu as pltpu
from jax.experimental.pallas import tpu_sc as plsc

N_NODES = 10000
D = 128
N_EDGES = 320000

NC = 2
NS = 16
NW = NC * NS

CHUNK = 128
ACC_ROWS = 10240
ROWS_PER_TILE = ACC_ROWS // NS
CHUNKS_PER_W = 80
EDGES_PAD = NW * CHUNK * CHUNKS_PER_W
N_DUMP = ACC_ROWS - N_NODES


def _sc_segment_sum(x, src_w, dst_w, zrows, zdeg):
    mesh = plsc.VectorSubcoreMesh(
        core_axis_name="c", subcore_axis_name="s", num_cores=NC, num_subcores=NS
    )

    @functools.partial(
        pl.kernel,
        mesh=mesh,
        out_type=(
            jax.ShapeDtypeStruct((NC, ACC_ROWS, D), jnp.float32),
            jax.ShapeDtypeStruct((NC, ACC_ROWS), jnp.float32),
        ),
        scratch_types=[
            pltpu.VMEM_SHARED((ACC_ROWS, D), jnp.float32),
            pltpu.VMEM_SHARED((ACC_ROWS,), jnp.float32),
            pltpu.VMEM((CHUNKS_PER_W // 2, CHUNK), jnp.int32),
            pltpu.VMEM((CHUNKS_PER_W // 2, CHUNK), jnp.int32),
            pltpu.VMEM((CHUNK, D), jnp.float32),
            pltpu.VMEM((CHUNK, D), jnp.float32),
            pltpu.VMEM((CHUNK,), jnp.float32),
            pltpu.SemaphoreType.DMA,
            pltpu.SemaphoreType.DMA,
        ],
    )
    def seg_sum(x_hbm, src_hbm, dst_hbm, zr_hbm, zd_hbm, part_hbm, deg_hbm,
                acc, dacc, srcv, dstv, rows0, rows1, ones, semA, semB):
        cid = lax.axis_index("c")
        sid = lax.axis_index("s")
        wid = sid * NC + cid
        base = sid * ROWS_PER_TILE

        pltpu.sync_copy(zr_hbm.at[pl.ds(base, ROWS_PER_TILE)],
                        acc.at[pl.ds(base, ROWS_PER_TILE)])
        pltpu.sync_copy(zd_hbm.at[pl.ds(base, ROWS_PER_TILE)],
                        dacc.at[pl.ds(base, ROWS_PER_TILE)])
        for i in range(CHUNK // 16):
            ones[pl.ds(i * 16, 16)] = jnp.full((16,), 1.0, jnp.float32)

        plsc.subcore_barrier()

        HALF = CHUNKS_PER_W // 2

        def gather(j, buf, sem):
            return pltpu.make_async_copy(x_hbm.at[srcv.at[j]], buf, sem)

        for p in range(2):
            pltpu.sync_copy(src_hbm.at[wid, pl.ds(p * HALF, HALF)], srcv)
            pltpu.sync_copy(dst_hbm.at[wid, pl.ds(p * HALF, HALF)], dstv)
            gather(0, rows0, semA).start()

            def body(g, _):
                j0 = 2 * g
                j1 = j0 + 1
                gather(j1, rows1, semB).start()
                gather(j0, rows0, semA).wait()
                pltpu.sync_copy(rows0, acc.at[dstv.at[j0]], add=True)
                pltpu.sync_copy(ones, dacc.at[dstv.at[j0]], add=True)

                @pl.when(g + 1 < HALF // 2)
                def _():
                    gather(j0 + 2, rows0, semA).start()

                gather(j1, rows1, semB).wait()
                pltpu.sync_copy(rows1, acc.at[dstv.at[j1]], add=True)
                pltpu.sync_copy(ones, dacc.at[dstv.at[j1]], add=True)
                return ()

            lax.fori_loop(0, HALF // 2, body, ())

        plsc.subcore_barrier()

        pltpu.sync_copy(acc.at[pl.ds(base, ROWS_PER_TILE)],
                        part_hbm.at[cid].at[pl.ds(base, ROWS_PER_TILE)])
        pltpu.sync_copy(dacc.at[pl.ds(base, ROWS_PER_TILE)],
                        deg_hbm.at[cid].at[pl.ds(base, ROWS_PER_TILE)])

    return seg_sum(x, src_w, dst_w, zrows, zdeg)


ROW_BLK = 1000


def _tc_body(part_ref0, part_ref1, x_ref, wl_ref, wr_ref, bl_ref, br_ref,
             d0_ref, d1_ref, o_ref):
    agg = part_ref0[0] + part_ref1[0]
    acc = jnp.dot(agg, wl_ref[...], preferred_element_type=jnp.float32)
    acc += jnp.dot(x_ref[...], wr_ref[...], preferred_element_type=jnp.float32)
    acc += (d0_ref[0] + d1_ref[0]) * bl_ref[...]
    acc += br_ref[...]
    o_ref[...] = jnp.maximum(acc, 0.0)


def _tc_epilogue(part, x, wl_t, wr_t, bl, br, degp):
    grid = (N_NODES // ROW_BLK,)
    blk = lambda i: (i, 0)
    full = lambda i: (0, 0)
    return pl.pallas_call(
        _tc_body,
        grid=grid,
        in_specs=[
            pl.BlockSpec((1, ROW_BLK, D), lambda i: (0, i, 0)),
            pl.BlockSpec((1, ROW_BLK, D), lambda i: (1, i, 0)),
            pl.BlockSpec((ROW_BLK, D), blk),
            pl.BlockSpec((D, D), full),
            pl.BlockSpec((D, D), full),
            pl.BlockSpec((1, D), full),
            pl.BlockSpec((1, D), full),
            pl.BlockSpec((1, ROW_BLK, 1), lambda i: (0, i, 0)),
            pl.BlockSpec((1, ROW_BLK, 1), lambda i: (1, i, 0)),
        ],
        out_specs=pl.BlockSpec((ROW_BLK, D), blk),
        out_shape=jax.ShapeDtypeStruct((N_NODES, D), jnp.float32),
    )(part, part, x, wl_t, wr_t, bl, br, degp, degp)


@jax.jit
def kernel(x, edge_index, W_lin, b_lin, W_root, b_root):
    src = edge_index[0].astype(jnp.int32)
    dst = edge_index[1].astype(jnp.int32)
    pad = EDGES_PAD - N_EDGES
    src_w = jnp.concatenate(
        [src, jnp.arange(pad, dtype=jnp.int32) % N_NODES])
    dst_w = jnp.concatenate(
        [dst, N_NODES + (jnp.arange(pad, dtype=jnp.int32) % N_DUMP)])
    src_w = src_w.reshape(NW, CHUNKS_PER_W, CHUNK)
    dst_w = dst_w.reshape(NW, CHUNKS_PER_W, CHUNK)
    zrows = jnp.zeros((ACC_ROWS, D), jnp.float32)
    zdeg = jnp.zeros((ACC_ROWS,), jnp.float32)

    part, degp = _sc_segment_sum(x, src_w, dst_w, zrows, zdeg)

    out = _tc_epilogue(
        part, x, W_lin.T, W_root.T,
        b_lin.reshape(1, D), b_root.reshape(1, D),
        degp.reshape(NC, ACC_ROWS, 1),
    )
    return out
